# baseline (speedup 1.0000x reference)
import functools

import jax
import jax.numpy as jnp
from jax import lax
from jax.experimental import pallas as pl
from jax.experimental.pallas import tpu as pltpu
from jax.experimental.pallas import tpu_sc as plsc

N = 10000
E = 160000
D = 512
NLAYER = 5

NC = 2
NS = 16
NW = NC * NS

CH = 128
NCHUNK = E // CH
CHA = 64
NCHA = E // CHA
QW = D // 4


def _sc_mesh():
    return plsc.VectorSubcoreMesh(
        core_axis_name="c", subcore_axis_name="s", num_cores=NC, num_subcores=NS
    )


def _sc_gather_add_body(x_hbm, e_hbm, idx_hbm, m_hbm, buf, gbuf, idx, sem):
    wid = lax.axis_index("s") * NC + lax.axis_index("c")

    def step(k, carry):
        ch = wid + NW * k

        @pl.when(ch < NCHA)
        def _():
            base = ch * CHA
            pltpu.sync_copy(idx_hbm.at[pl.ds(base, CHA)], idx)
            cp = pltpu.async_copy(x_hbm.at[idx], gbuf, sem)
            pltpu.sync_copy(e_hbm.at[pl.ds(base, CHA)], buf)
            cp.wait()

            def row(r, c2):
                def col(cc, c3):
                    sl = pl.ds(cc * 16, 16)
                    buf[r, sl] = buf[r, sl] + gbuf[r, sl]
                    return c3

                return lax.fori_loop(0, D // 16, col, c2)

            lax.fori_loop(0, CHA, row, 0)
            for qq in range(4):
                pltpu.sync_copy(
                    buf.at[:, pl.ds(qq * QW, QW)],
                    m_hbm.at[qq, pl.ds(base, CHA)],
                )

        return carry

    nsteps = NCHA // NW + 1
    lax.fori_loop(0, nsteps, step, 0)


def _sc_gather_add(x, e, idx):
    k = pl.kernel(
        _sc_gather_add_body,
        out_type=jax.ShapeDtypeStruct((4, E, QW), jnp.float32),
        mesh=_sc_mesh(),
        scratch_types=[
            pltpu.VMEM((CHA, D), jnp.float32),
            pltpu.VMEM((CHA, D), jnp.float32),
            pltpu.VMEM((CHA,), jnp.int32),
            pltpu.SemaphoreType.DMA,
        ],
    )
    return k(x, e, idx)


def _sc_gather_add_relu_body(x_hbm, m_hbm, idx_hbm, e_hbm, buf, gbuf, idx, sem):
    wid = lax.axis_index("s") * NC + lax.axis_index("c")

    def step(k, carry):
        ch = wid + NW * k

        @pl.when(ch < NCHA)
        def _():
            base = ch * CHA
            pltpu.sync_copy(idx_hbm.at[pl.ds(base, CHA)], idx)
            cp = pltpu.async_copy(x_hbm.at[idx], gbuf, sem)
            for qq in range(4):
                pltpu.sync_copy(
                    m_hbm.at[qq, pl.ds(base, CHA)],
                    buf.at[:, pl.ds(qq * QW, QW)],
                )
            cp.wait()

            def row(r, c2):
                def col(cc, c3):
                    sl = pl.ds(cc * 16, 16)
                    buf[r, sl] = jnp.maximum(buf[r, sl] + gbuf[r, sl], 0.0)
                    return c3

                return lax.fori_loop(0, D // 16, col, c2)

            lax.fori_loop(0, CHA, row, 0)
            pltpu.sync_copy(buf, e_hbm.at[pl.ds(base, CHA)])

        return carry

    nsteps = NCHA // NW + 1
    lax.fori_loop(0, nsteps, step, 0)


def _sc_gather_add_relu(x, m, idx):
    k = pl.kernel(
        _sc_gather_add_relu_body,
        out_type=jax.ShapeDtypeStruct((E, D), jnp.float32),
        mesh=_sc_mesh(),
        scratch_types=[
            pltpu.VMEM((CHA, D), jnp.float32),
            pltpu.VMEM((CHA, D), jnp.float32),
            pltpu.VMEM((CHA,), jnp.int32),
            pltpu.SemaphoreType.DMA,
        ],
    )
    return k(x, m, idx)


ZR = 80
NZCH = N // ZR


def _sc_segsum_body(m_hbm, idx_hbm, agg_hbm, agg_sh, zbuf, mbuf, idx):
    cid = lax.axis_index("c")
    sid = lax.axis_index("s")

    def zrow(r, carry):
        def zcol(cc, c3):
            zbuf[r, pl.ds(cc * 16, 16)] = jnp.zeros((16,), jnp.float32)
            return c3

        return lax.fori_loop(0, QW // 16, zcol, carry)

    lax.fori_loop(0, ZR, zrow, 0)

    for j in range(2):
        q = cid * 2 + j

        def zcopy(t, carry):
            ch = sid + NS * t

            @pl.when(ch < NZCH)
            def _():
                pltpu.sync_copy(zbuf, agg_sh.at[pl.ds(ch * ZR, ZR)])

            return carry

        lax.fori_loop(0, NZCH // NS + 1, zcopy, 0)
        plsc.subcore_barrier()

        def step(k, carry):
            ch = sid + NS * k

            @pl.when(ch < NCHUNK)
            def _():
                base = ch * CH
                pltpu.sync_copy(m_hbm.at[q, pl.ds(base, CH)], mbuf)
                pltpu.sync_copy(idx_hbm.at[pl.ds(base, CH)], idx)
                pltpu.sync_copy(mbuf, agg_sh.at[idx], add=True)

            return carry

        lax.fori_loop(0, NCHUNK // NS + 1, step, 0)
        plsc.subcore_barrier()

        def wb(t, carry):
            ch = sid + NS * t

            @pl.when(ch < NZCH)
            def _():
                pltpu.sync_copy(
                    agg_sh.at[pl.ds(ch * ZR, ZR)],
                    agg_hbm.at[q, pl.ds(ch * ZR, ZR)],
                )

            return carry

        lax.fori_loop(0, NZCH // NS + 1, wb, 0)
        plsc.subcore_barrier()


def _sc_segsum(m, idx):
    k = pl.kernel(
        _sc_segsum_body,
        out_type=jax.ShapeDtypeStruct((4, N, QW), jnp.float32),
        mesh=_sc_mesh(),
        scratch_types=[
            pltpu.VMEM_SHARED((N, QW), jnp.float32),
            pltpu.VMEM((ZR, QW), jnp.float32),
            pltpu.VMEM((CH, QW), jnp.float32),
            pltpu.VMEM((CH,), jnp.int32),
        ],
    )
    return k(m, idx)


def _mlp_body(nw, softmax, *refs):
    x_ref = refs[0]
    w_refs = refs[1 : 1 + 2 * nw]
    o_ref = refs[1 + 2 * nw]
    h = x_ref[...]
    for i in range(nw):
        w = w_refs[2 * i][...]
        b = w_refs[2 * i + 1][...]
        h = jnp.dot(h, w, preferred_element_type=jnp.float32) + b
        if i < nw - 1:
            h = jnp.maximum(h, 0.0)
    if softmax:
        h = h - jnp.max(h, axis=1, keepdims=True)
        h = jnp.exp(h)
        h = h / jnp.sum(h, axis=1, keepdims=True)
    o_ref[...] = h


def _mlp_apply_tc(params, x, block_rows, softmax=False):
    rows = x.shape[0]
    nw = len(params)
    dout = params[-1][0].shape[1]
    grid = rows // block_rows
    in_specs = [pl.BlockSpec((block_rows, x.shape[1]), lambda i: (i, 0))]
    ops = [x]
    for (w, b) in params:
        in_specs.append(pl.BlockSpec(w.shape, lambda i: (0, 0)))
        in_specs.append(pl.BlockSpec((1, b.shape[0]), lambda i: (0, 0)))
        ops.append(w)
        ops.append(b.reshape(1, -1))
    return pl.pallas_call(
        functools.partial(_mlp_body, nw, softmax),
        grid=(grid,),
        in_specs=in_specs,
        out_specs=pl.BlockSpec((block_rows, dout), lambda i: (i, 0)),
        out_shape=jax.ShapeDtypeStruct((rows, dout), jnp.float32),
    )(*ops)


def _xupdate_body(x_ref, a_ref, w_ref, b_ref, o_ref):
    a = a_ref[...]
    agg = jnp.concatenate([a[0], a[1], a[2], a[3]], axis=1)
    h = x_ref[...] + agg
    h = jnp.dot(h, w_ref[...], preferred_element_type=jnp.float32) + b_ref[...]
    o_ref[...] = jnp.maximum(h, 0.0)


def _xupdate_tc(x, agg, w, b, block_rows=1000):
    grid = N // block_rows
    return pl.pallas_call(
        _xupdate_body,
        grid=(grid,),
        in_specs=[
            pl.BlockSpec((block_rows, D), lambda i: (i, 0)),
            pl.BlockSpec((4, block_rows, QW), lambda i: (0, i, 0)),
            pl.BlockSpec((D, D), lambda i: (0, 0)),
            pl.BlockSpec((1, D), lambda i: (0, 0)),
        ],
        out_specs=pl.BlockSpec((block_rows, D), lambda i: (i, 0)),
        out_shape=jax.ShapeDtypeStruct((N, D), jnp.float32),
    )(x, agg, w, b.reshape(1, -1))


def kernel(obj_onehot, pred_onehot, edge_index, node_enc, edge_enc, gnn_Wn,
           gnn_bn, node_dec, edge_dec, node_prototype, edge_prototype):
    src = edge_index[0].astype(jnp.int32)
    dst = edge_index[1].astype(jnp.int32)

    x = _mlp_apply_tc(node_enc, obj_onehot, 1000)
    e = _mlp_apply_tc(edge_enc, pred_onehot, 2000)

    for l in range(NLAYER):
        m = _sc_gather_add(x, e, src)
        agg = _sc_segsum(m, dst)
        x = _xupdate_tc(x, agg, gnn_Wn[l], gnn_bn[l])
        e = _sc_gather_add_relu(x, m, dst)

    node_output = _mlp_apply_tc(node_dec, x, 1000, softmax=True)
    edge_output = _mlp_apply_tc(edge_dec, e, 2000, softmax=True)
    return (node_output, edge_output, x, e, node_prototype, edge_prototype,
            obj_onehot, pred_onehot)


# speedup vs baseline: 2.3936x; 2.3936x over previous
import functools

import jax
import jax.numpy as jnp
from jax import lax
from jax.experimental import pallas as pl
from jax.experimental.pallas import tpu as pltpu
from jax.experimental.pallas import tpu_sc as plsc

N = 10000
E = 160000
D = 512
NLAYER = 5

NC = 2
NS = 16
NW = NC * NS

CH = 128
NCHUNK = E // CH
CHA = 40
NCHA = E // CHA
PT = NCHA // NW
QW = D // 4


def _sc_mesh():
    return plsc.VectorSubcoreMesh(
        core_axis_name="c", subcore_axis_name="s", num_cores=NC, num_subcores=NS
    )


def _pipelined_gather_combine(x_hbm, row_hbm, idx_hbm, out_hbm, bufs, combine):
    wid = lax.axis_index("s") * NC + lax.axis_index("c")
    c0 = wid * PT

    def issue(ch, b):
        buf, gbuf, idx, sg = bufs[b]
        base = ch * CHA
        pltpu.sync_copy(idx_hbm.at[pl.ds(base, CHA)], idx)
        pltpu.async_copy(x_hbm.at[idx], gbuf, sg)
        pltpu.async_copy(row_hbm.at[pl.ds(base, CHA)], buf, sg)

    def process(ch, b):
        buf, gbuf, idx, sg = bufs[b]
        pltpu.make_async_copy(x_hbm.at[pl.ds(0, CHA)], gbuf, sg).wait()
        pltpu.make_async_copy(x_hbm.at[pl.ds(0, CHA)], buf, sg).wait()

        def row(r, c2):
            for cc in range(D // 16):
                sl = pl.ds(cc * 16, 16)
                buf[r, sl] = combine(buf[r, sl], gbuf[r, sl])
            return c2

        lax.fori_loop(0, CHA, row, 0)
        pltpu.sync_copy(buf, out_hbm.at[pl.ds(ch * CHA, CHA)])

    issue(c0, 0)

    def g_step(g, carry):
        l = c0 + 2 * g
        issue(l + 1, 1)
        process(l, 0)
        issue(l + 2, 0)
        process(l + 1, 1)
        return carry

    lax.fori_loop(0, (PT - 1) // 2, g_step, 0)
    process(c0 + PT - 1, 0)


_GATHER_SCRATCH = [
    pltpu.VMEM((CHA, D), jnp.float32),
    pltpu.VMEM((CHA, D), jnp.float32),
    pltpu.VMEM((CHA,), jnp.int32),
    pltpu.SemaphoreType.DMA,
    pltpu.VMEM((CHA, D), jnp.float32),
    pltpu.VMEM((CHA, D), jnp.float32),
    pltpu.VMEM((CHA,), jnp.int32),
    pltpu.SemaphoreType.DMA,
]


def _sc_gather_add_body(x_hbm, e_hbm, idx_hbm, m_hbm,
                        b0, g0, i0, s0, b1, g1, i1, s1):
    bufs = ((b0, g0, i0, s0), (b1, g1, i1, s1))
    _pipelined_gather_combine(x_hbm, e_hbm, idx_hbm, m_hbm, bufs,
                              lambda a, b: a + b)


def _sc_gather_add(x, e, idx):
    k = pl.kernel(
        _sc_gather_add_body,
        out_type=jax.ShapeDtypeStruct((E, D), jnp.float32),
        mesh=_sc_mesh(),
        scratch_types=_GATHER_SCRATCH,
    )
    return k(x, e, idx)


def _sc_gather_add_relu_body(x_hbm, m_hbm, idx_hbm, e_hbm,
                             b0, g0, i0, s0, b1, g1, i1, s1):
    bufs = ((b0, g0, i0, s0), (b1, g1, i1, s1))
    _pipelined_gather_combine(x_hbm, m_hbm, idx_hbm, e_hbm, bufs,
                              lambda a, b: jnp.maximum(a + b, 0.0))


def _sc_gather_add_relu(x, m, idx):
    k = pl.kernel(
        _sc_gather_add_relu_body,
        out_type=jax.ShapeDtypeStruct((E, D), jnp.float32),
        mesh=_sc_mesh(),
        scratch_types=_GATHER_SCRATCH,
    )
    return k(x, m, idx)


ZR = 80
NZCH = N // ZR


def _sc_segsum_body(m_hbm, idx_hbm, agg_hbm, agg_sh, zbuf, mbuf, idx):
    cid = lax.axis_index("c")
    sid = lax.axis_index("s")

    def zrow(r, carry):
        def zcol(cc, c3):
            zbuf[r, pl.ds(cc * 16, 16)] = jnp.zeros((16,), jnp.float32)
            return c3

        return lax.fori_loop(0, QW // 16, zcol, carry)

    lax.fori_loop(0, ZR, zrow, 0)

    for j in range(2):
        def zcopy(t, carry):
            ch = sid + NS * t

            @pl.when(ch < NZCH)
            def _():
                pltpu.sync_copy(zbuf, agg_sh.at[pl.ds(ch * ZR, ZR)])

            return carry

        lax.fori_loop(0, NZCH // NS + 1, zcopy, 0)
        plsc.subcore_barrier()

        for cs in range(NC):
            q = cs * 2 + j

            @pl.when(cid == cs)
            def _(q=q):
                def step(k, carry):
                    ch = sid + NS * k

                    @pl.when(ch < NCHUNK)
                    def _():
                        base = ch * CH
                        pltpu.sync_copy(
                            m_hbm.at[pl.ds(base, CH), pl.ds(q * QW, QW)], mbuf
                        )
                        pltpu.sync_copy(idx_hbm.at[pl.ds(base, CH)], idx)
                        pltpu.sync_copy(mbuf, agg_sh.at[idx], add=True)

                    return carry

                lax.fori_loop(0, NCHUNK // NS + 1, step, 0)

        plsc.subcore_barrier()

        for cs in range(NC):
            q = cs * 2 + j

            @pl.when(cid == cs)
            def _(q=q):
                def wb(t, carry):
                    ch = sid + NS * t

                    @pl.when(ch < NZCH)
                    def _():
                        pltpu.sync_copy(
                            agg_sh.at[pl.ds(ch * ZR, ZR)],
                            agg_hbm.at[q, pl.ds(ch * ZR, ZR)],
                        )

                    return carry

                lax.fori_loop(0, NZCH // NS + 1, wb, 0)

        plsc.subcore_barrier()


def _sc_segsum(m, idx):
    k = pl.kernel(
        _sc_segsum_body,
        out_type=jax.ShapeDtypeStruct((4, N, QW), jnp.float32),
        mesh=_sc_mesh(),
        scratch_types=[
            pltpu.VMEM_SHARED((N, QW), jnp.float32),
            pltpu.VMEM((ZR, QW), jnp.float32),
            pltpu.VMEM((CH, QW), jnp.float32),
            pltpu.VMEM((CH,), jnp.int32),
        ],
    )
    return k(m, idx)


def _mlp_body(nw, softmax, *refs):
    x_ref = refs[0]
    w_refs = refs[1 : 1 + 2 * nw]
    o_ref = refs[1 + 2 * nw]
    h = x_ref[...]
    for i in range(nw):
        w = w_refs[2 * i][...]
        b = w_refs[2 * i + 1][...]
        h = jnp.dot(h, w, preferred_element_type=jnp.float32) + b
        if i < nw - 1:
            h = jnp.maximum(h, 0.0)
    if softmax:
        h = h - jnp.max(h, axis=1, keepdims=True)
        h = jnp.exp(h)
        h = h / jnp.sum(h, axis=1, keepdims=True)
    o_ref[...] = h


def _mlp_apply_tc(params, x, block_rows, softmax=False):
    rows = x.shape[0]
    nw = len(params)
    dout = params[-1][0].shape[1]
    grid = rows // block_rows
    in_specs = [pl.BlockSpec((block_rows, x.shape[1]), lambda i: (i, 0))]
    ops = [x]
    for (w, b) in params:
        in_specs.append(pl.BlockSpec(w.shape, lambda i: (0, 0)))
        in_specs.append(pl.BlockSpec((1, b.shape[0]), lambda i: (0, 0)))
        ops.append(w)
        ops.append(b.reshape(1, -1))
    return pl.pallas_call(
        functools.partial(_mlp_body, nw, softmax),
        grid=(grid,),
        in_specs=in_specs,
        out_specs=pl.BlockSpec((block_rows, dout), lambda i: (i, 0)),
        out_shape=jax.ShapeDtypeStruct((rows, dout), jnp.float32),
    )(*ops)


def _xupdate_body(x_ref, a_ref, w_ref, b_ref, o_ref):
    a = a_ref[...]
    agg = jnp.concatenate([a[0], a[1], a[2], a[3]], axis=1)
    h = x_ref[...] + agg
    h = jnp.dot(h, w_ref[...], preferred_element_type=jnp.float32) + b_ref[...]
    o_ref[...] = jnp.maximum(h, 0.0)


def _xupdate_tc(x, agg, w, b, block_rows=1000):
    grid = N // block_rows
    return pl.pallas_call(
        _xupdate_body,
        grid=(grid,),
        in_specs=[
            pl.BlockSpec((block_rows, D), lambda i: (i, 0)),
            pl.BlockSpec((4, block_rows, QW), lambda i: (0, i, 0)),
            pl.BlockSpec((D, D), lambda i: (0, 0)),
            pl.BlockSpec((1, D), lambda i: (0, 0)),
        ],
        out_specs=pl.BlockSpec((block_rows, D), lambda i: (i, 0)),
        out_shape=jax.ShapeDtypeStruct((N, D), jnp.float32),
    )(x, agg, w, b.reshape(1, -1))


def kernel(obj_onehot, pred_onehot, edge_index, node_enc, edge_enc, gnn_Wn,
           gnn_bn, node_dec, edge_dec, node_prototype, edge_prototype):
    src = edge_index[0].astype(jnp.int32)
    dst = edge_index[1].astype(jnp.int32)

    x = _mlp_apply_tc(node_enc, obj_onehot, 1000)
    e = _mlp_apply_tc(edge_enc, pred_onehot, 2000)

    for l in range(NLAYER):
        m = _sc_gather_add(x, e, src)
        agg = _sc_segsum(m, dst)
        x = _xupdate_tc(x, agg, gnn_Wn[l], gnn_bn[l])
        e = _sc_gather_add_relu(x, m, dst)

    node_output = _mlp_apply_tc(node_dec, x, 1000, softmax=True)
    edge_output = _mlp_apply_tc(edge_dec, e, 2000, softmax=True)
    return (node_output, edge_output, x, e, node_prototype, edge_prototype,
            obj_onehot, pred_onehot)


# speedup vs baseline: 3.0266x; 1.2645x over previous
import functools

import jax
import jax.numpy as jnp
from jax import lax
from jax.experimental import pallas as pl
from jax.experimental.pallas import tpu as pltpu
from jax.experimental.pallas import tpu_sc as plsc

N = 10000
E = 160000
D = 512
NLAYER = 5

NC = 2
NS = 16
NW = NC * NS

CH = 128
NCHUNK = E // CH
CHA = 40
NCHA = E // CHA
PT = NCHA // NW
QW = D // 4


def _sc_mesh():
    return plsc.VectorSubcoreMesh(
        core_axis_name="c", subcore_axis_name="s", num_cores=NC, num_subcores=NS
    )


def _pipelined_gather_combine(x_hbm, row_hbm, idx_hbm, out_hbm, bufs, idxall,
                              combine):
    wid = lax.axis_index("s") * NC + lax.axis_index("c")
    c0 = wid * PT
    pltpu.sync_copy(idx_hbm.at[pl.ds(c0 * CHA, PT * CHA)], idxall)

    def issue(ch, b):
        buf, gbuf, sg, sw = bufs[b]

        @pl.when(ch >= c0 + 2)
        def _():
            pltpu.make_async_copy(
                row_hbm.at[pl.ds(0, CHA)], buf, sw
            ).wait()

        loc = (ch - c0) * CHA
        pltpu.async_copy(x_hbm.at[idxall.at[pl.ds(loc, CHA)]], gbuf, sg)
        pltpu.async_copy(row_hbm.at[pl.ds(ch * CHA, CHA)], buf, sg)

    def process(ch, b):
        buf, gbuf, sg, sw = bufs[b]
        pltpu.make_async_copy(x_hbm.at[pl.ds(0, CHA)], gbuf, sg).wait()
        pltpu.make_async_copy(x_hbm.at[pl.ds(0, CHA)], buf, sg).wait()

        def row(r, c2):
            for cc in range(D // 16):
                sl = pl.ds(cc * 16, 16)
                buf[r, sl] = combine(buf[r, sl], gbuf[r, sl])
            return c2

        lax.fori_loop(0, CHA, row, 0)
        pltpu.async_copy(buf, out_hbm.at[pl.ds(ch * CHA, CHA)], sw)

    issue(c0, 0)

    def g_step(g, carry):
        l = c0 + 2 * g
        issue(l + 1, 1)
        process(l, 0)
        issue(l + 2, 0)
        process(l + 1, 1)
        return carry

    lax.fori_loop(0, (PT - 1) // 2, g_step, 0)
    process(c0 + PT - 1, 0)
    for b in range(2):
        buf = bufs[b][0]
        sw = bufs[b][3]
        pltpu.make_async_copy(row_hbm.at[pl.ds(0, CHA)], buf, sw).wait()


_GATHER_SCRATCH = [
    pltpu.VMEM((CHA, D), jnp.float32),
    pltpu.VMEM((CHA, D), jnp.float32),
    pltpu.SemaphoreType.DMA,
    pltpu.SemaphoreType.DMA,
    pltpu.VMEM((CHA, D), jnp.float32),
    pltpu.VMEM((CHA, D), jnp.float32),
    pltpu.SemaphoreType.DMA,
    pltpu.SemaphoreType.DMA,
    pltpu.VMEM((PT * CHA,), jnp.int32),
]


def _sc_gather_add_body(x_hbm, e_hbm, idx_hbm, m_hbm,
                        b0, g0, s0, w0, b1, g1, s1, w1, idxall):
    bufs = ((b0, g0, s0, w0), (b1, g1, s1, w1))
    _pipelined_gather_combine(x_hbm, e_hbm, idx_hbm, m_hbm, bufs, idxall,
                              lambda a, b: a + b)


def _sc_gather_add(x, e, idx):
    k = pl.kernel(
        _sc_gather_add_body,
        out_type=jax.ShapeDtypeStruct((E, D), jnp.float32),
        mesh=_sc_mesh(),
        scratch_types=_GATHER_SCRATCH,
    )
    return k(x, e, idx)


def _sc_gather_add_relu_body(x_hbm, m_hbm, idx_hbm, e_hbm,
                             b0, g0, s0, w0, b1, g1, s1, w1, idxall):
    bufs = ((b0, g0, s0, w0), (b1, g1, s1, w1))
    _pipelined_gather_combine(x_hbm, m_hbm, idx_hbm, e_hbm, bufs, idxall,
                              lambda a, b: jnp.maximum(a + b, 0.0))


def _sc_gather_add_relu(x, m, idx):
    k = pl.kernel(
        _sc_gather_add_relu_body,
        out_type=jax.ShapeDtypeStruct((E, D), jnp.float32),
        mesh=_sc_mesh(),
        scratch_types=_GATHER_SCRATCH,
    )
    return k(x, m, idx)


ZR = 80
NZCH = N // ZR


def _sc_segsum_body(m_hbm, idx_hbm, agg_hbm, agg_sh, zbuf,
                    mb0, ib0, sr0, ss0, mb1, ib1, sr1, ss1):
    cid = lax.axis_index("c")
    sid = lax.axis_index("s")
    rings = ((mb0, ib0, sr0, ss0), (mb1, ib1, sr1, ss1))

    def zrow(r, carry):
        def zcol(cc, c3):
            zbuf[r, pl.ds(cc * 16, 16)] = jnp.zeros((16,), jnp.float32)
            return c3

        return lax.fori_loop(0, QW // 16, zcol, carry)

    lax.fori_loop(0, ZR, zrow, 0)

    for j in range(2):
        def zcopy(t, carry):
            ch = sid + NS * t

            @pl.when(ch < NZCH)
            def _():
                pltpu.sync_copy(zbuf, agg_sh.at[pl.ds(ch * ZR, ZR)])

            return carry

        lax.fori_loop(0, NZCH // NS + 1, zcopy, 0)
        plsc.subcore_barrier()

        for cs in range(NC):
            q = cs * 2 + j

            @pl.when(cid == cs)
            def _(q=q):
                def issue(k, b):
                    mb, ib, sr, ss = rings[b]
                    ch = sid + NS * k

                    @pl.when(ch < NCHUNK)
                    def _():
                        base = ch * CH
                        pltpu.async_copy(
                            m_hbm.at[pl.ds(base, CH), pl.ds(q * QW, QW)],
                            mb, sr,
                        )
                        pltpu.async_copy(idx_hbm.at[pl.ds(base, CH)], ib, sr)

                def process(k, b):
                    mb, ib, sr, ss = rings[b]
                    ch = sid + NS * k

                    @pl.when(ch < NCHUNK)
                    def _():
                        pltpu.make_async_copy(
                            m_hbm.at[pl.ds(0, CH), pl.ds(0, QW)], mb, sr
                        ).wait()
                        pltpu.make_async_copy(
                            idx_hbm.at[pl.ds(0, CH)], ib, sr
                        ).wait()
                        pltpu.sync_copy(mb, agg_sh.at[ib], add=True)

                issue(0, 0)

                def g_step(g, carry):
                    issue(2 * g + 1, 1)
                    process(2 * g, 0)
                    issue(2 * g + 2, 0)
                    process(2 * g + 1, 1)
                    return carry

                lax.fori_loop(0, (NCHUNK // NS + 1) // 2 + 1, g_step, 0)

        plsc.subcore_barrier()

        for cs in range(NC):
            q = cs * 2 + j

            @pl.when(cid == cs)
            def _(q=q):
                def wb(t, carry):
                    ch = sid + NS * t

                    @pl.when(ch < NZCH)
                    def _():
                        pltpu.sync_copy(
                            agg_sh.at[pl.ds(ch * ZR, ZR)],
                            agg_hbm.at[q, pl.ds(ch * ZR, ZR)],
                        )

                    return carry

                lax.fori_loop(0, NZCH // NS + 1, wb, 0)

        plsc.subcore_barrier()


def _sc_segsum(m, idx):
    k = pl.kernel(
        _sc_segsum_body,
        out_type=jax.ShapeDtypeStruct((4, N, QW), jnp.float32),
        mesh=_sc_mesh(),
        scratch_types=[
            pltpu.VMEM_SHARED((N, QW), jnp.float32),
            pltpu.VMEM((ZR, QW), jnp.float32),
            pltpu.VMEM((CH, QW), jnp.float32),
            pltpu.VMEM((CH,), jnp.int32),
            pltpu.SemaphoreType.DMA,
            pltpu.SemaphoreType.DMA,
            pltpu.VMEM((CH, QW), jnp.float32),
            pltpu.VMEM((CH,), jnp.int32),
            pltpu.SemaphoreType.DMA,
            pltpu.SemaphoreType.DMA,
        ],
    )
    return k(m, idx)


def _mlp_body(nw, softmax, *refs):
    x_ref = refs[0]
    w_refs = refs[1 : 1 + 2 * nw]
    o_ref = refs[1 + 2 * nw]
    h = x_ref[...]
    for i in range(nw):
        w = w_refs[2 * i][...]
        b = w_refs[2 * i + 1][...]
        h = jnp.dot(h, w, preferred_element_type=jnp.float32) + b
        if i < nw - 1:
            h = jnp.maximum(h, 0.0)
    if softmax:
        h = h - jnp.max(h, axis=1, keepdims=True)
        h = jnp.exp(h)
        h = h / jnp.sum(h, axis=1, keepdims=True)
    o_ref[...] = h


def _mlp_apply_tc(params, x, block_rows, softmax=False):
    rows = x.shape[0]
    nw = len(params)
    dout = params[-1][0].shape[1]
    grid = rows // block_rows
    in_specs = [pl.BlockSpec((block_rows, x.shape[1]), lambda i: (i, 0))]
    ops = [x]
    for (w, b) in params:
        in_specs.append(pl.BlockSpec(w.shape, lambda i: (0, 0)))
        in_specs.append(pl.BlockSpec((1, b.shape[0]), lambda i: (0, 0)))
        ops.append(w)
        ops.append(b.reshape(1, -1))
    return pl.pallas_call(
        functools.partial(_mlp_body, nw, softmax),
        grid=(grid,),
        in_specs=in_specs,
        out_specs=pl.BlockSpec((block_rows, dout), lambda i: (i, 0)),
        out_shape=jax.ShapeDtypeStruct((rows, dout), jnp.float32),
    )(*ops)


def _xupdate_body(x_ref, a_ref, w_ref, b_ref, o_ref):
    a = a_ref[...]
    agg = jnp.concatenate([a[0], a[1], a[2], a[3]], axis=1)
    h = x_ref[...] + agg
    h = jnp.dot(h, w_ref[...], preferred_element_type=jnp.float32) + b_ref[...]
    o_ref[...] = jnp.maximum(h, 0.0)


def _xupdate_tc(x, agg, w, b, block_rows=1000):
    grid = N // block_rows
    return pl.pallas_call(
        _xupdate_body,
        grid=(grid,),
        in_specs=[
            pl.BlockSpec((block_rows, D), lambda i: (i, 0)),
            pl.BlockSpec((4, block_rows, QW), lambda i: (0, i, 0)),
            pl.BlockSpec((D, D), lambda i: (0, 0)),
            pl.BlockSpec((1, D), lambda i: (0, 0)),
        ],
        out_specs=pl.BlockSpec((block_rows, D), lambda i: (i, 0)),
        out_shape=jax.ShapeDtypeStruct((N, D), jnp.float32),
    )(x, agg, w, b.reshape(1, -1))


def kernel(obj_onehot, pred_onehot, edge_index, node_enc, edge_enc, gnn_Wn,
           gnn_bn, node_dec, edge_dec, node_prototype, edge_prototype):
    src = edge_index[0].astype(jnp.int32)
    dst = edge_index[1].astype(jnp.int32)

    x = _mlp_apply_tc(node_enc, obj_onehot, 1000)
    e = _mlp_apply_tc(edge_enc, pred_onehot, 2000)

    for l in range(NLAYER):
        m = _sc_gather_add(x, e, src)
        agg = _sc_segsum(m, dst)
        x = _xupdate_tc(x, agg, gnn_Wn[l], gnn_bn[l])
        e = _sc_gather_add_relu(x, m, dst)

    node_output = _mlp_apply_tc(node_dec, x, 1000, softmax=True)
    edge_output = _mlp_apply_tc(edge_dec, e, 2000, softmax=True)
    return (node_output, edge_output, x, e, node_prototype, edge_prototype,
            obj_onehot, pred_onehot)


# speedup vs baseline: 3.2481x; 1.0732x over previous
import functools

import jax
import jax.numpy as jnp
from jax import lax
from jax.experimental import pallas as pl
from jax.experimental.pallas import tpu as pltpu
from jax.experimental.pallas import tpu_sc as plsc

N = 10000
E = 160000
D = 512
NLAYER = 5

NC = 2
NS = 16
NW = NC * NS

CH = 64
NCHUNK = E // CH
PB = NCHUNK // NS
NEXTRA = NCHUNK - PB * NS
CHA = 40
NCHA = E // CHA
PT = NCHA // NW
QW = D // 4
ZR = 40
NZCH = N // ZR


def _sc_mesh():
    return plsc.VectorSubcoreMesh(
        core_axis_name="c", subcore_axis_name="s", num_cores=NC, num_subcores=NS
    )


def _sc_msg_segsum_body(xq_hbm, eq_hbm, sidx_hbm, didx_hbm, mq_hbm, agg_hbm,
                        agg_sh, zbuf, sidxall,
                        eb0, gb0, di0, si0, sw0, eb1, gb1, di1, si1, sw1):
    cid = lax.axis_index("c")
    sid = lax.axis_index("s")
    rings = ((eb0, gb0, di0, si0, sw0), (eb1, gb1, di1, si1, sw1))
    c0 = sid * PB
    pltpu.sync_copy(sidx_hbm.at[pl.ds(c0 * CH, PB * CH)], sidxall)

    def zrow(r, carry):
        for cc in range(QW // 16):
            zbuf[r, pl.ds(cc * 16, 16)] = jnp.zeros((16,), jnp.float32)
        return carry

    lax.fori_loop(0, ZR, zrow, 0)

    for j in range(2):
        def zcopy(t, carry):
            ch = sid + NS * t

            @pl.when(ch < NZCH)
            def _():
                pltpu.sync_copy(zbuf, agg_sh.at[pl.ds(ch * ZR, ZR)])

            return carry

        lax.fori_loop(0, NZCH // NS + 1, zcopy, 0)
        plsc.subcore_barrier()

        for cs in range(NC):
            q = cs * 2 + j

            @pl.when(cid == cs)
            def _(q=q):
                def issue(k, b):
                    eb, gb, di, si, sw = rings[b]

                    @pl.when(k < PB)
                    def _():
                        @pl.when(k >= 2)
                        def _():
                            pltpu.make_async_copy(
                                eb, mq_hbm.at[q, pl.ds(0, CH)], sw
                            ).wait()

                        base = (c0 + k) * CH
                        pltpu.async_copy(eq_hbm.at[q, pl.ds(base, CH)], eb, si)
                        pltpu.async_copy(
                            xq_hbm.at[q].at[sidxall.at[pl.ds(k * CH, CH)]],
                            gb, si,
                        )
                        pltpu.async_copy(didx_hbm.at[pl.ds(base, CH)], di, si)

                def process(k, b):
                    eb, gb, di, si, sw = rings[b]
                    pltpu.make_async_copy(
                        eq_hbm.at[q, pl.ds(0, CH)], eb, si
                    ).wait()
                    pltpu.make_async_copy(
                        eq_hbm.at[q, pl.ds(0, CH)], gb, si
                    ).wait()
                    pltpu.make_async_copy(
                        didx_hbm.at[pl.ds(0, CH)], di, si
                    ).wait()

                    def row(r, c2):
                        for cc in range(QW // 16):
                            sl = pl.ds(cc * 16, 16)
                            eb[r, sl] = eb[r, sl] + gb[r, sl]
                        return c2

                    lax.fori_loop(0, CH, row, 0)
                    base = (c0 + k) * CH
                    pltpu.async_copy(eb, mq_hbm.at[q, pl.ds(base, CH)], sw)
                    pltpu.sync_copy(eb, agg_sh.at[di], add=True)

                issue(jnp.int32(0), 0)

                def g_step(g, carry):
                    l = 2 * g
                    issue(l + 1, 1)
                    process(l, 0)
                    issue(l + 2, 0)
                    process(l + 1, 1)
                    return carry

                lax.fori_loop(0, PB // 2, g_step, 0)
                for b in range(2):
                    eb = rings[b][0]
                    sw = rings[b][4]
                    pltpu.make_async_copy(
                        eb, mq_hbm.at[q, pl.ds(0, CH)], sw
                    ).wait()

                @pl.when(sid < NEXTRA)
                def _():
                    ch = NS * PB + sid
                    base = ch * CH
                    pltpu.sync_copy(eq_hbm.at[q, pl.ds(base, CH)], eb0)
                    pltpu.sync_copy(sidx_hbm.at[pl.ds(base, CH)], di0)
                    pltpu.async_copy(
                        xq_hbm.at[q].at[di0], gb0, si0
                    ).wait()

                    def rowx(r, c2):
                        for cc in range(QW // 16):
                            sl = pl.ds(cc * 16, 16)
                            eb0[r, sl] = eb0[r, sl] + gb0[r, sl]
                        return c2

                    lax.fori_loop(0, CH, rowx, 0)
                    pltpu.sync_copy(eb0, mq_hbm.at[q, pl.ds(base, CH)])
                    pltpu.sync_copy(didx_hbm.at[pl.ds(base, CH)], di0)
                    pltpu.sync_copy(eb0, agg_sh.at[di0], add=True)

        plsc.subcore_barrier()

        for cs in range(NC):
            q = cs * 2 + j

            @pl.when(cid == cs)
            def _(q=q):
                def wb(t, carry):
                    ch = sid + NS * t

                    @pl.when(ch < NZCH)
                    def _():
                        pltpu.sync_copy(
                            agg_sh.at[pl.ds(ch * ZR, ZR)],
                            agg_hbm.at[q, pl.ds(ch * ZR, ZR)],
                        )

                    return carry

                lax.fori_loop(0, NZCH // NS + 1, wb, 0)

        plsc.subcore_barrier()


def _sc_msg_segsum(xq, eq, sidx, didx):
    k = pl.kernel(
        _sc_msg_segsum_body,
        out_type=(
            jax.ShapeDtypeStruct((4, E, QW), jnp.float32),
            jax.ShapeDtypeStruct((4, N, QW), jnp.float32),
        ),
        mesh=_sc_mesh(),
        scratch_types=[
            pltpu.VMEM_SHARED((N, QW), jnp.float32),
            pltpu.VMEM((ZR, QW), jnp.float32),
            pltpu.VMEM((PB * CH,), jnp.int32),
            pltpu.VMEM((CH, QW), jnp.float32),
            pltpu.VMEM((CH, QW), jnp.float32),
            pltpu.VMEM((CH,), jnp.int32),
            pltpu.SemaphoreType.DMA,
            pltpu.SemaphoreType.DMA,
            pltpu.VMEM((CH, QW), jnp.float32),
            pltpu.VMEM((CH, QW), jnp.float32),
            pltpu.VMEM((CH,), jnp.int32),
            pltpu.SemaphoreType.DMA,
            pltpu.SemaphoreType.DMA,
        ],
    )
    return k(xq, eq, sidx, didx)


def _sc_edge_update_body(row_major_out, x_hbm, mq_hbm, idx_hbm, out_hbm,
                         b0, g0, s0, w0, b1, g1, s1, w1, idxall):
    bufs = ((b0, g0, s0, w0), (b1, g1, s1, w1))
    wid = lax.axis_index("s") * NC + lax.axis_index("c")
    c0 = wid * PT
    pltpu.sync_copy(idx_hbm.at[pl.ds(c0 * CHA, PT * CHA)], idxall)

    def write_out(buf, ch, sw):
        if row_major_out:
            pltpu.async_copy(buf, out_hbm.at[pl.ds(ch * CHA, CHA)], sw)
        else:
            for qq in range(4):
                pltpu.async_copy(
                    buf.at[:, pl.ds(qq * QW, QW)],
                    out_hbm.at[qq, pl.ds(ch * CHA, CHA)], sw,
                )

    def drain_out(buf, sw):
        pltpu.make_async_copy(x_hbm.at[pl.ds(0, CHA)], buf, sw).wait()

    def issue(ch, b):
        buf, gbuf, sg, sw = bufs[b]

        @pl.when(ch >= c0 + 2)
        def _():
            drain_out(buf, sw)

        loc = (ch - c0) * CHA
        pltpu.async_copy(x_hbm.at[idxall.at[pl.ds(loc, CHA)]], gbuf, sg)
        for qq in range(4):
            pltpu.async_copy(
                mq_hbm.at[qq, pl.ds(ch * CHA, CHA)],
                buf.at[:, pl.ds(qq * QW, QW)], sg,
            )

    def process(ch, b):
        buf, gbuf, sg, sw = bufs[b]
        pltpu.make_async_copy(x_hbm.at[pl.ds(0, CHA)], gbuf, sg).wait()
        pltpu.make_async_copy(x_hbm.at[pl.ds(0, CHA)], buf, sg).wait()

        def row(r, c2):
            for cc in range(D // 16):
                sl = pl.ds(cc * 16, 16)
                buf[r, sl] = jnp.maximum(buf[r, sl] + gbuf[r, sl], 0.0)
            return c2

        lax.fori_loop(0, CHA, row, 0)
        write_out(buf, ch, sw)

    issue(c0, 0)

    def g_step(g, carry):
        l = c0 + 2 * g
        issue(l + 1, 1)
        process(l, 0)
        issue(l + 2, 0)
        process(l + 1, 1)
        return carry

    lax.fori_loop(0, (PT - 1) // 2, g_step, 0)
    process(c0 + PT - 1, 0)
    for b in range(2):
        drain_out(bufs[b][0], bufs[b][3])


_EDGE_SCRATCH = [
    pltpu.VMEM((CHA, D), jnp.float32),
    pltpu.VMEM((CHA, D), jnp.float32),
    pltpu.SemaphoreType.DMA,
    pltpu.SemaphoreType.DMA,
    pltpu.VMEM((CHA, D), jnp.float32),
    pltpu.VMEM((CHA, D), jnp.float32),
    pltpu.SemaphoreType.DMA,
    pltpu.SemaphoreType.DMA,
    pltpu.VMEM((PT * CHA,), jnp.int32),
]


def _sc_edge_update(x, mq, idx, row_major_out):
    shape = (E, D) if row_major_out else (4, E, QW)
    k = pl.kernel(
        functools.partial(_sc_edge_update_body, row_major_out),
        out_type=jax.ShapeDtypeStruct(shape, jnp.float32),
        mesh=_sc_mesh(),
        scratch_types=_EDGE_SCRATCH,
    )
    return k(x, mq, idx)


def _mlp_body(nw, softmax, qin, qout, *refs):
    x_ref = refs[0]
    w_refs = refs[1 : 1 + 2 * nw]
    if qin:
        a = x_ref[...]
        h = jnp.concatenate([a[0], a[1], a[2], a[3]], axis=1)
    else:
        h = x_ref[...]
    for i in range(nw):
        w = w_refs[2 * i][...]
        b = w_refs[2 * i + 1][...]
        h = jnp.dot(h, w, preferred_element_type=jnp.float32) + b
        if i < nw - 1:
            h = jnp.maximum(h, 0.0)
    if softmax:
        h = h - jnp.max(h, axis=1, keepdims=True)
        h = jnp.exp(h)
        h = h / jnp.sum(h, axis=1, keepdims=True)
    if qout:
        o_ref = refs[1 + 2 * nw]
        for qq in range(4):
            o_ref[qq] = h[:, qq * QW:(qq + 1) * QW]
    else:
        refs[1 + 2 * nw][...] = h


def _mlp_apply_tc(params, x, block_rows, softmax=False, qin=False, qout=False):
    rows = x.shape[1] if qin else x.shape[0]
    nw = len(params)
    dout = params[-1][0].shape[1]
    grid = rows // block_rows
    if qin:
        in_specs = [pl.BlockSpec((4, block_rows, QW), lambda i: (0, i, 0))]
    else:
        in_specs = [pl.BlockSpec((block_rows, x.shape[1]), lambda i: (i, 0))]
    ops = [x]
    for (w, b) in params:
        in_specs.append(pl.BlockSpec(w.shape, lambda i: (0, 0)))
        in_specs.append(pl.BlockSpec((1, b.shape[0]), lambda i: (0, 0)))
        ops.append(w)
        ops.append(b.reshape(1, -1))
    if qout:
        out_specs = pl.BlockSpec((4, block_rows, QW), lambda i: (0, i, 0))
        out_shape = jax.ShapeDtypeStruct((4, rows, QW), jnp.float32)
    else:
        out_specs = pl.BlockSpec((block_rows, dout), lambda i: (i, 0))
        out_shape = jax.ShapeDtypeStruct((rows, dout), jnp.float32)
    return pl.pallas_call(
        functools.partial(_mlp_body, nw, softmax, qin, qout),
        grid=(grid,),
        in_specs=in_specs,
        out_specs=out_specs,
        out_shape=out_shape,
    )(*ops)


def _xupdate_body(x_ref, a_ref, w_ref, b_ref, o_ref, oq_ref):
    a = a_ref[...]
    agg = jnp.concatenate([a[0], a[1], a[2], a[3]], axis=1)
    h = x_ref[...] + agg
    h = jnp.dot(h, w_ref[...], preferred_element_type=jnp.float32) + b_ref[...]
    h = jnp.maximum(h, 0.0)
    o_ref[...] = h
    for qq in range(4):
        oq_ref[qq] = h[:, qq * QW:(qq + 1) * QW]


def _xupdate_tc(x, agg, w, b, block_rows=1000):
    grid = N // block_rows
    return pl.pallas_call(
        _xupdate_body,
        grid=(grid,),
        in_specs=[
            pl.BlockSpec((block_rows, D), lambda i: (i, 0)),
            pl.BlockSpec((4, block_rows, QW), lambda i: (0, i, 0)),
            pl.BlockSpec((D, D), lambda i: (0, 0)),
            pl.BlockSpec((1, D), lambda i: (0, 0)),
        ],
        out_specs=[
            pl.BlockSpec((block_rows, D), lambda i: (i, 0)),
            pl.BlockSpec((4, block_rows, QW), lambda i: (0, i, 0)),
        ],
        out_shape=[
            jax.ShapeDtypeStruct((N, D), jnp.float32),
            jax.ShapeDtypeStruct((4, N, QW), jnp.float32),
        ],
    )(x, agg, w, b.reshape(1, -1))


def _split_q_body(x_ref, oq_ref):
    h = x_ref[...]
    for qq in range(4):
        oq_ref[qq] = h[:, qq * QW:(qq + 1) * QW]


def _split_q_tc(x, block_rows=1000):
    grid = x.shape[0] // block_rows
    return pl.pallas_call(
        _split_q_body,
        grid=(grid,),
        in_specs=[pl.BlockSpec((block_rows, D), lambda i: (i, 0))],
        out_specs=pl.BlockSpec((4, block_rows, QW), lambda i: (0, i, 0)),
        out_shape=jax.ShapeDtypeStruct((4, x.shape[0], QW), jnp.float32),
    )(x)


def kernel(obj_onehot, pred_onehot, edge_index, node_enc, edge_enc, gnn_Wn,
           gnn_bn, node_dec, edge_dec, node_prototype, edge_prototype):
    src = edge_index[0].astype(jnp.int32)
    dst = edge_index[1].astype(jnp.int32)

    x = _mlp_apply_tc(node_enc, obj_onehot, 1000)
    xq = _split_q_tc(x)
    eq = _mlp_apply_tc(edge_enc, pred_onehot, 2000, qout=True)

    e = None
    for l in range(NLAYER):
        mq, agg = _sc_msg_segsum(xq, eq, src, dst)
        x, xq = _xupdate_tc(x, agg, gnn_Wn[l], gnn_bn[l])
        last = l == NLAYER - 1
        out = _sc_edge_update(x, mq, dst, row_major_out=last)
        if last:
            e = out
        else:
            eq = out

    node_output = _mlp_apply_tc(node_dec, x, 1000, softmax=True)
    edge_output = _mlp_apply_tc(edge_dec, e, 2000, softmax=True)
    return (node_output, edge_output, x, e, node_prototype, edge_prototype,
            obj_onehot, pred_onehot)


# speedup vs baseline: 3.2494x; 1.0004x over previous
import functools

import jax
import jax.numpy as jnp
from jax import lax
from jax.experimental import pallas as pl
from jax.experimental.pallas import tpu as pltpu
from jax.experimental.pallas import tpu_sc as plsc

N = 10000
E = 160000
D = 512
NLAYER = 5

NC = 2
NS = 16
NW = NC * NS

CH = 64
NCHUNK = E // CH
PB = NCHUNK // NS
NEXTRA = NCHUNK - PB * NS
CHA = 40
NCHA = E // CHA
PT = NCHA // NW
QW = D // 4
ZR = 40
NZCH = N // ZR


def _sc_mesh():
    return plsc.VectorSubcoreMesh(
        core_axis_name="c", subcore_axis_name="s", num_cores=NC, num_subcores=NS
    )


def _sc_msg_segsum_body(xq_hbm, eq_hbm, sidx_hbm, didx_hbm, mq_hbm, agg_hbm,
                        agg_sh, zbuf, sidxall,
                        eb0, gb0, di0, si0, sw0, ss0,
                        eb1, gb1, di1, si1, sw1, ss1):
    cid = lax.axis_index("c")
    sid = lax.axis_index("s")
    rings = ((eb0, gb0, di0, si0, sw0, ss0), (eb1, gb1, di1, si1, sw1, ss1))
    c0 = sid * PB
    pltpu.sync_copy(sidx_hbm.at[pl.ds(c0 * CH, PB * CH)], sidxall)

    def zrow(r, carry):
        for cc in range(QW // 16):
            zbuf[r, pl.ds(cc * 16, 16)] = jnp.zeros((16,), jnp.float32)
        return carry

    lax.fori_loop(0, ZR, zrow, 0)

    for j in range(2):
        def zcopy(t, carry):
            ch = sid + NS * t

            @pl.when(ch < NZCH)
            def _():
                pltpu.sync_copy(zbuf, agg_sh.at[pl.ds(ch * ZR, ZR)])

            return carry

        lax.fori_loop(0, NZCH // NS + 1, zcopy, 0)
        plsc.subcore_barrier()

        for cs in range(NC):
            q = cs * 2 + j

            @pl.when(cid == cs)
            def _(q=q):
                def issue(k, b):
                    eb, gb, di, si, sw, ss = rings[b]

                    @pl.when(k < PB)
                    def _():
                        @pl.when(k >= 2)
                        def _():
                            pltpu.make_async_copy(
                                eb, mq_hbm.at[q, pl.ds(0, CH)], sw
                            ).wait()
                            pltpu.make_async_copy(
                                eb, agg_sh.at[di], ss
                            ).wait()

                        base = (c0 + k) * CH
                        pltpu.async_copy(eq_hbm.at[q, pl.ds(base, CH)], eb, si)
                        pltpu.async_copy(
                            xq_hbm.at[q].at[sidxall.at[pl.ds(k * CH, CH)]],
                            gb, si,
                        )
                        pltpu.async_copy(didx_hbm.at[pl.ds(base, CH)], di, si)

                def process(k, b):
                    eb, gb, di, si, sw, ss = rings[b]
                    pltpu.make_async_copy(
                        eq_hbm.at[q, pl.ds(0, CH)], eb, si
                    ).wait()
                    pltpu.make_async_copy(
                        eq_hbm.at[q, pl.ds(0, CH)], gb, si
                    ).wait()
                    pltpu.make_async_copy(
                        didx_hbm.at[pl.ds(0, CH)], di, si
                    ).wait()

                    def row(r, c2):
                        for cc in range(QW // 16):
                            sl = pl.ds(cc * 16, 16)
                            eb[r, sl] = eb[r, sl] + gb[r, sl]
                        return c2

                    lax.fori_loop(0, CH, row, 0)
                    base = (c0 + k) * CH
                    pltpu.async_copy(eb, mq_hbm.at[q, pl.ds(base, CH)], sw)
                    pltpu.async_copy(eb, agg_sh.at[di], ss, add=True)

                issue(jnp.int32(0), 0)

                def g_step(g, carry):
                    l = 2 * g
                    issue(l + 1, 1)
                    process(l, 0)
                    issue(l + 2, 0)
                    process(l + 1, 1)
                    return carry

                lax.fori_loop(0, PB // 2, g_step, 0)
                for b in range(2):
                    eb, gb, di, si, sw, ss = rings[b]
                    pltpu.make_async_copy(
                        eb, mq_hbm.at[q, pl.ds(0, CH)], sw
                    ).wait()
                    pltpu.make_async_copy(eb, agg_sh.at[di], ss).wait()

                @pl.when(sid < NEXTRA)
                def _():
                    ch = NS * PB + sid
                    base = ch * CH
                    pltpu.sync_copy(eq_hbm.at[q, pl.ds(base, CH)], eb0)
                    pltpu.sync_copy(sidx_hbm.at[pl.ds(base, CH)], di0)
                    pltpu.async_copy(
                        xq_hbm.at[q].at[di0], gb0, si0
                    ).wait()

                    def rowx(r, c2):
                        for cc in range(QW // 16):
                            sl = pl.ds(cc * 16, 16)
                            eb0[r, sl] = eb0[r, sl] + gb0[r, sl]
                        return c2

                    lax.fori_loop(0, CH, rowx, 0)
                    pltpu.sync_copy(eb0, mq_hbm.at[q, pl.ds(base, CH)])
                    pltpu.sync_copy(didx_hbm.at[pl.ds(base, CH)], di0)
                    pltpu.sync_copy(eb0, agg_sh.at[di0], add=True)

        plsc.subcore_barrier()

        for cs in range(NC):
            q = cs * 2 + j

            @pl.when(cid == cs)
            def _(q=q):
                def wb(t, carry):
                    ch = sid + NS * t

                    @pl.when(ch < NZCH)
                    def _():
                        pltpu.sync_copy(
                            agg_sh.at[pl.ds(ch * ZR, ZR)],
                            agg_hbm.at[q, pl.ds(ch * ZR, ZR)],
                        )

                    return carry

                lax.fori_loop(0, NZCH // NS + 1, wb, 0)

        plsc.subcore_barrier()


def _sc_msg_segsum(xq, eq, sidx, didx):
    k = pl.kernel(
        _sc_msg_segsum_body,
        out_type=(
            jax.ShapeDtypeStruct((4, E, QW), jnp.float32),
            jax.ShapeDtypeStruct((4, N, QW), jnp.float32),
        ),
        mesh=_sc_mesh(),
        scratch_types=[
            pltpu.VMEM_SHARED((N, QW), jnp.float32),
            pltpu.VMEM((ZR, QW), jnp.float32),
            pltpu.VMEM((PB * CH,), jnp.int32),
            pltpu.VMEM((CH, QW), jnp.float32),
            pltpu.VMEM((CH, QW), jnp.float32),
            pltpu.VMEM((CH,), jnp.int32),
            pltpu.SemaphoreType.DMA,
            pltpu.SemaphoreType.DMA,
            pltpu.SemaphoreType.DMA,
            pltpu.VMEM((CH, QW), jnp.float32),
            pltpu.VMEM((CH, QW), jnp.float32),
            pltpu.VMEM((CH,), jnp.int32),
            pltpu.SemaphoreType.DMA,
            pltpu.SemaphoreType.DMA,
            pltpu.SemaphoreType.DMA,
        ],
    )
    return k(xq, eq, sidx, didx)


def _sc_edge_update_body(row_major_out, x_hbm, mq_hbm, idx_hbm, out_hbm,
                         b0, g0, s0, w0, b1, g1, s1, w1, idxall):
    bufs = ((b0, g0, s0, w0), (b1, g1, s1, w1))
    wid = lax.axis_index("s") * NC + lax.axis_index("c")
    c0 = wid * PT
    pltpu.sync_copy(idx_hbm.at[pl.ds(c0 * CHA, PT * CHA)], idxall)

    def write_out(buf, ch, sw):
        if row_major_out:
            pltpu.async_copy(buf, out_hbm.at[pl.ds(ch * CHA, CHA)], sw)
        else:
            for qq in range(4):
                pltpu.async_copy(
                    buf.at[:, pl.ds(qq * QW, QW)],
                    out_hbm.at[qq, pl.ds(ch * CHA, CHA)], sw,
                )

    def drain_out(buf, sw):
        pltpu.make_async_copy(x_hbm.at[pl.ds(0, CHA)], buf, sw).wait()

    def issue(ch, b):
        buf, gbuf, sg, sw = bufs[b]

        @pl.when(ch >= c0 + 2)
        def _():
            drain_out(buf, sw)

        loc = (ch - c0) * CHA
        pltpu.async_copy(x_hbm.at[idxall.at[pl.ds(loc, CHA)]], gbuf, sg)
        for qq in range(4):
            pltpu.async_copy(
                mq_hbm.at[qq, pl.ds(ch * CHA, CHA)],
                buf.at[:, pl.ds(qq * QW, QW)], sg,
            )

    def process(ch, b):
        buf, gbuf, sg, sw = bufs[b]
        pltpu.make_async_copy(x_hbm.at[pl.ds(0, CHA)], gbuf, sg).wait()
        pltpu.make_async_copy(x_hbm.at[pl.ds(0, CHA)], buf, sg).wait()

        def row(r, c2):
            for cc in range(D // 16):
                sl = pl.ds(cc * 16, 16)
                buf[r, sl] = jnp.maximum(buf[r, sl] + gbuf[r, sl], 0.0)
            return c2

        lax.fori_loop(0, CHA, row, 0)
        write_out(buf, ch, sw)

    issue(c0, 0)

    def g_step(g, carry):
        l = c0 + 2 * g
        issue(l + 1, 1)
        process(l, 0)
        issue(l + 2, 0)
        process(l + 1, 1)
        return carry

    lax.fori_loop(0, (PT - 1) // 2, g_step, 0)
    process(c0 + PT - 1, 0)
    for b in range(2):
        drain_out(bufs[b][0], bufs[b][3])


_EDGE_SCRATCH = [
    pltpu.VMEM((CHA, D), jnp.float32),
    pltpu.VMEM((CHA, D), jnp.float32),
    pltpu.SemaphoreType.DMA,
    pltpu.SemaphoreType.DMA,
    pltpu.VMEM((CHA, D), jnp.float32),
    pltpu.VMEM((CHA, D), jnp.float32),
    pltpu.SemaphoreType.DMA,
    pltpu.SemaphoreType.DMA,
    pltpu.VMEM((PT * CHA,), jnp.int32),
]


def _sc_edge_update(x, mq, idx, row_major_out):
    shape = (E, D) if row_major_out else (4, E, QW)
    k = pl.kernel(
        functools.partial(_sc_edge_update_body, row_major_out),
        out_type=jax.ShapeDtypeStruct(shape, jnp.float32),
        mesh=_sc_mesh(),
        scratch_types=_EDGE_SCRATCH,
    )
    return k(x, mq, idx)


def _mlp_body(nw, softmax, qin, qout, *refs):
    x_ref = refs[0]
    w_refs = refs[1 : 1 + 2 * nw]
    if qin:
        a = x_ref[...]
        h = jnp.concatenate([a[0], a[1], a[2], a[3]], axis=1)
    else:
        h = x_ref[...]
    for i in range(nw):
        w = w_refs[2 * i][...]
        b = w_refs[2 * i + 1][...]
        h = jnp.dot(h, w, preferred_element_type=jnp.float32) + b
        if i < nw - 1:
            h = jnp.maximum(h, 0.0)
    if softmax:
        h = h - jnp.max(h, axis=1, keepdims=True)
        h = jnp.exp(h)
        h = h / jnp.sum(h, axis=1, keepdims=True)
    if qout:
        o_ref = refs[1 + 2 * nw]
        for qq in range(4):
            o_ref[qq] = h[:, qq * QW:(qq + 1) * QW]
    else:
        refs[1 + 2 * nw][...] = h


def _mlp_apply_tc(params, x, block_rows, softmax=False, qin=False, qout=False):
    rows = x.shape[1] if qin else x.shape[0]
    nw = len(params)
    dout = params[-1][0].shape[1]
    grid = rows // block_rows
    if qin:
        in_specs = [pl.BlockSpec((4, block_rows, QW), lambda i: (0, i, 0))]
    else:
        in_specs = [pl.BlockSpec((block_rows, x.shape[1]), lambda i: (i, 0))]
    ops = [x]
    for (w, b) in params:
        in_specs.append(pl.BlockSpec(w.shape, lambda i: (0, 0)))
        in_specs.append(pl.BlockSpec((1, b.shape[0]), lambda i: (0, 0)))
        ops.append(w)
        ops.append(b.reshape(1, -1))
    if qout:
        out_specs = pl.BlockSpec((4, block_rows, QW), lambda i: (0, i, 0))
        out_shape = jax.ShapeDtypeStruct((4, rows, QW), jnp.float32)
    else:
        out_specs = pl.BlockSpec((block_rows, dout), lambda i: (i, 0))
        out_shape = jax.ShapeDtypeStruct((rows, dout), jnp.float32)
    return pl.pallas_call(
        functools.partial(_mlp_body, nw, softmax, qin, qout),
        grid=(grid,),
        in_specs=in_specs,
        out_specs=out_specs,
        out_shape=out_shape,
    )(*ops)


def _xupdate_body(x_ref, a_ref, w_ref, b_ref, o_ref, oq_ref):
    a = a_ref[...]
    agg = jnp.concatenate([a[0], a[1], a[2], a[3]], axis=1)
    h = x_ref[...] + agg
    h = jnp.dot(h, w_ref[...], preferred_element_type=jnp.float32) + b_ref[...]
    h = jnp.maximum(h, 0.0)
    o_ref[...] = h
    for qq in range(4):
        oq_ref[qq] = h[:, qq * QW:(qq + 1) * QW]


def _xupdate_tc(x, agg, w, b, block_rows=1000):
    grid = N // block_rows
    return pl.pallas_call(
        _xupdate_body,
        grid=(grid,),
        in_specs=[
            pl.BlockSpec((block_rows, D), lambda i: (i, 0)),
            pl.BlockSpec((4, block_rows, QW), lambda i: (0, i, 0)),
            pl.BlockSpec((D, D), lambda i: (0, 0)),
            pl.BlockSpec((1, D), lambda i: (0, 0)),
        ],
        out_specs=[
            pl.BlockSpec((block_rows, D), lambda i: (i, 0)),
            pl.BlockSpec((4, block_rows, QW), lambda i: (0, i, 0)),
        ],
        out_shape=[
            jax.ShapeDtypeStruct((N, D), jnp.float32),
            jax.ShapeDtypeStruct((4, N, QW), jnp.float32),
        ],
    )(x, agg, w, b.reshape(1, -1))


def _split_q_body(x_ref, oq_ref):
    h = x_ref[...]
    for qq in range(4):
        oq_ref[qq] = h[:, qq * QW:(qq + 1) * QW]


def _split_q_tc(x, block_rows=1000):
    grid = x.shape[0] // block_rows
    return pl.pallas_call(
        _split_q_body,
        grid=(grid,),
        in_specs=[pl.BlockSpec((block_rows, D), lambda i: (i, 0))],
        out_specs=pl.BlockSpec((4, block_rows, QW), lambda i: (0, i, 0)),
        out_shape=jax.ShapeDtypeStruct((4, x.shape[0], QW), jnp.float32),
    )(x)


def kernel(obj_onehot, pred_onehot, edge_index, node_enc, edge_enc, gnn_Wn,
           gnn_bn, node_dec, edge_dec, node_prototype, edge_prototype):
    src = edge_index[0].astype(jnp.int32)
    dst = edge_index[1].astype(jnp.int32)

    x = _mlp_apply_tc(node_enc, obj_onehot, 1000)
    xq = _split_q_tc(x)
    eq = _mlp_apply_tc(edge_enc, pred_onehot, 2000, qout=True)

    e = None
    for l in range(NLAYER):
        mq, agg = _sc_msg_segsum(xq, eq, src, dst)
        x, xq = _xupdate_tc(x, agg, gnn_Wn[l], gnn_bn[l])
        last = l == NLAYER - 1
        out = _sc_edge_update(x, mq, dst, row_major_out=last)
        if last:
            e = out
        else:
            eq = out

    node_output = _mlp_apply_tc(node_dec, x, 1000, softmax=True)
    edge_output = _mlp_apply_tc(edge_dec, e, 2000, softmax=True)
    return (node_output, edge_output, x, e, node_prototype, edge_prototype,
            obj_onehot, pred_onehot)
